# baseline (speedup 1.0000x reference)
import jax
import jax.numpy as jnp
from jax.experimental import pallas as pl


def _shift_body(x_ref, frow_ref, fcol_ref, out_ref):
    x = x_ref[0]
    c2 = x.shape[0]
    ch = c2 // 2
    fmr = x[:ch]
    lat = x[ch:]
    frow = frow_ref[0]
    fcol = fcol_ref[0]
    n = lat.shape[1]

    lat_t = lat.T
    norm = jnp.sqrt(jnp.sum(lat_t * lat_t, axis=1, keepdims=True)) + 1e-8
    latn = lat_t / norm
    sim = jax.lax.dot_general(
        latn, latn, (((1,), (1,)), ((), ())),
        preferred_element_type=jnp.float32,
        precision=jax.lax.Precision.DEFAULT)
    sim = jnp.where(frow > 0, jnp.float32(-1e9), sim)
    rowmax = jnp.max(sim, axis=1, keepdims=True)
    kiota = jax.lax.broadcasted_iota(jnp.int32, (n, n), 1)
    idx = jnp.min(jnp.where(sim == rowmax, kiota, n), axis=1,
                  keepdims=True)
    niota = jax.lax.broadcasted_iota(jnp.int32, (n, 1), 0)
    sel = jnp.where(fcol > 0, idx, niota)
    onehot = (sel == kiota).astype(jnp.float32)
    shifted = jax.lax.dot_general(
        fmr, onehot, (((1,), (1,)), ((), ())),
        preferred_element_type=jnp.float32,
        precision=jax.lax.Precision.HIGHEST)
    out_ref[0, :c2] = x
    out_ref[0, c2:] = shifted


def kernel(input, mask):
    b, c, h, w = input.shape
    ch = c // 2
    n = h * w
    x = input.reshape(b, c, n)
    frow = (mask.reshape(1, 1, n) >= 1).astype(jnp.int32)
    fcol = frow.reshape(1, n, 1)

    out = pl.pallas_call(
        _shift_body,
        grid=(b,),
        in_specs=[
            pl.BlockSpec((1, c, n), lambda i: (i, 0, 0)),
            pl.BlockSpec((1, 1, n), lambda i: (0, 0, 0)),
            pl.BlockSpec((1, n, 1), lambda i: (0, 0, 0)),
        ],
        out_specs=pl.BlockSpec((1, c + ch, n), lambda i: (i, 0, 0)),
        out_shape=jax.ShapeDtypeStruct((b, c + ch, n), jnp.float32),
    )(x, frow, fcol)

    return out.reshape(b, c + ch, h, w)


# speedup vs baseline: 1.1332x; 1.1332x over previous
import jax
import jax.numpy as jnp
from jax.experimental import pallas as pl
from jax.experimental.pallas import tpu as pltpu


def _shift_body(x_ref, frow_ref, fcol_ref, out_ref):
    x = x_ref[0]
    c2 = x.shape[0]
    ch = c2 // 2
    fmr = x[:ch]
    lat = x[ch:]
    frow = frow_ref[0]
    fcol = fcol_ref[0]
    n = lat.shape[1]

    lat_t = lat.T
    norm = jnp.sqrt(jnp.sum(lat_t * lat_t, axis=1, keepdims=True)) + 1e-8
    latn = lat_t / norm
    sim = jax.lax.dot_general(
        latn, latn, (((1,), (1,)), ((), ())),
        preferred_element_type=jnp.float32,
        precision=jax.lax.Precision.DEFAULT)
    sim = jnp.where(frow > 0, jnp.float32(-1e9), sim)
    rowmax = jnp.max(sim, axis=1, keepdims=True)
    kiota = jax.lax.broadcasted_iota(jnp.int32, (n, n), 1)
    idx = jnp.min(jnp.where(sim == rowmax, kiota, n), axis=1,
                  keepdims=True)
    niota = jax.lax.broadcasted_iota(jnp.int32, (n, 1), 0)
    sel = jnp.where(fcol > 0, idx, niota)
    onehot = (sel == kiota).astype(jnp.bfloat16)
    f0 = fmr.astype(jnp.bfloat16)
    r1 = fmr - f0.astype(jnp.float32)
    f1 = r1.astype(jnp.bfloat16)
    f2 = (r1 - f1.astype(jnp.float32)).astype(jnp.bfloat16)

    def _pass(f):
        return jax.lax.dot_general(
            f, onehot, (((1,), (1,)), ((), ())),
            preferred_element_type=jnp.float32)

    shifted = (_pass(f0) + _pass(f1)) + _pass(f2)
    out_ref[0, :c2] = x
    out_ref[0, c2:] = shifted


def kernel(input, mask):
    b, c, h, w = input.shape
    ch = c // 2
    n = h * w
    x = input.reshape(b, c, n)
    frow = (mask.reshape(1, 1, n) >= 1).astype(jnp.int32)
    fcol = frow.reshape(1, n, 1)

    out = pl.pallas_call(
        _shift_body,
        grid=(b,),
        in_specs=[
            pl.BlockSpec((1, c, n), lambda i: (i, 0, 0)),
            pl.BlockSpec((1, 1, n), lambda i: (0, 0, 0)),
            pl.BlockSpec((1, n, 1), lambda i: (0, 0, 0)),
        ],
        out_specs=pl.BlockSpec((1, c + ch, n), lambda i: (i, 0, 0)),
        out_shape=jax.ShapeDtypeStruct((b, c + ch, n), jnp.float32),
        compiler_params=pltpu.CompilerParams(
            dimension_semantics=("parallel",)),
    )(x, frow, fcol)

    return out.reshape(b, c + ch, h, w)
